# baseline (speedup 1.0000x reference)
import functools

import jax
import jax.numpy as jnp
from jax import lax
from jax.experimental import pallas as pl
from jax.experimental.pallas import tpu as pltpu
from jax.experimental.pallas import tpu_sc as plsc

B = 16384
CTX = 20
D = 64
L = 16
NC = 2
NS = 16
NW = NC * NS
ROWS_PER_W = B // NW
CHUNK = 32
N_CHUNKS = ROWS_PER_W // CHUNK
IDX_PER_CHUNK = CHUNK * CTX
G = 128
NG = IDX_PER_CHUNK // G
VOCAB = 1000000
VP = VOCAB // 2

FULL_TILES = VOCAB // 128
TAIL_COLS = VOCAB - FULL_TILES * 128
SB = 4
N_SB = FULL_TILES // SB
SB_COLS = SB * 128
SB_PAIRS = SB_COLS // 2
SB_PER_W = -(-N_SB // NW)


def _make_reformat():
    mesh = plsc.VectorSubcoreMesh(
        core_axis_name="c", subcore_axis_name="s", num_cores=NC, num_subcores=NS
    )

    @functools.partial(
        pl.kernel,
        out_type=jax.ShapeDtypeStruct((VP, 2 * D), jnp.float32),
        mesh=mesh,
        compiler_params=pltpu.CompilerParams(
            needs_layout_passes=False, disable_bounds_checks=True),
        scratch_types=[
            pltpu.VMEM((D, SB_COLS + 3), jnp.float32),
            pltpu.VMEM((D, SB_COLS + 3), jnp.float32),
            pltpu.VMEM((SB_PAIRS // 2, 2 * D), jnp.float32),
            pltpu.VMEM((SB_PAIRS // 2, 2 * D), jnp.float32),
            pltpu.VMEM((D, TAIL_COLS), jnp.float32),
            pltpu.VMEM((TAIL_COLS // 2, 2 * D), jnp.float32),
            pltpu.SemaphoreType.DMA,
            pltpu.SemaphoreType.DMA,
            pltpu.SemaphoreType.DMA,
            pltpu.SemaphoreType.DMA,
        ],
    )
    def reformat_kernel(tt_hbm, pairs_hbm, inb0, inb1, outb0, outb1, tin, tout,
                        isem0, isem1, osem0, osem1):
        wid = lax.axis_index("s") * NC + lax.axis_index("c")
        sb0 = wid * SB_PER_W
        n_sb = jnp.maximum(0, jnp.minimum(SB_PER_W, N_SB - sb0))
        inbs, outbs = (inb0, inb1), (outb0, outb1)
        isems, osems = (isem0, isem1), (osem0, osem1)

        lanes = lax.iota(jnp.int32, L)
        rowk = [(16 * (k % 4) + lanes) for k in range(8)]
        UNROLL = 4

        def start_in(bi, par):
            pltpu.async_copy(
                tt_hbm.at[:, pl.ds((sb0 + bi) * SB_COLS, SB_COLS)],
                inbs[par].at[:, pl.ds(0, SB_COLS)], isems[par])

        def wait_in(par):
            pltpu.make_async_copy(
                tt_hbm.at[:, pl.ds(0, SB_COLS)],
                inbs[par].at[:, pl.ds(0, SB_COLS)], isems[par]).wait()

        HP = SB_PAIRS // 2

        def start_out(bi, half):
            pltpu.async_copy(
                outbs[half],
                pairs_hbm.at[pl.ds((sb0 + bi) * SB_PAIRS + half * HP, HP)],
                osems[half])

        def wait_out(half):
            pltpu.make_async_copy(
                outbs[half], pairs_hbm.at[pl.ds(0, HP)], osems[half]).wait()

        def transpose_rows(n_rows, q0, src, dst):
            @plsc.parallel_loop(0, n_rows, step=1, unroll=UNROLL)
            def _rows(q):
                c0 = jnp.full((L,), 2 * (q + q0), jnp.int32)
                c1 = c0 + 1
                for k in range(8):
                    vals = plsc.load_gather(src, [rowk[k], (c0, c1)[k // 4]])
                    dst[q, pl.ds(k * L, L)] = vals

        @pl.when(n_sb > 0)
        def _prime0():
            start_in(0, 0)

        @pl.when(n_sb > 1)
        def _prime1():
            start_in(1, 1)

        def outer(bi, carry):
            for par in range(2):

                @pl.when((bi % 2 == par) & (bi < n_sb))
                def _():
                    wait_in(par)
                    for half in range(2):

                        @pl.when(bi >= 1)
                        def _drain():
                            wait_out(half)

                        transpose_rows(HP, half * HP, inbs[par], outbs[half])
                        start_out(bi, half)

                    @pl.when(bi + 2 < n_sb)
                    def _next():
                        start_in(bi + 2, par)
            return carry

        lax.fori_loop(0, SB_PER_W, outer, 0)

        @pl.when(n_sb > 0)
        def _drain_both():
            wait_out(0)
            wait_out(1)

        @pl.when(wid == NW - 1)
        def _tail():
            c0 = FULL_TILES * 128
            pltpu.sync_copy(tt_hbm.at[:, pl.ds(c0, TAIL_COLS)], tin)

            def trow(q, carry):
                tc0, tc1 = carry
                for k in range(8):
                    vals = plsc.load_gather(tin, [rowk[k], (tc0, tc1)[k // 4]])
                    tout[q, pl.ds(k * L, L)] = vals
                return (tc0 + 2, tc1 + 2)

            zero = jnp.zeros((L,), jnp.int32)
            lax.fori_loop(0, TAIL_COLS // 2, trow, (zero, zero + 1))
            pltpu.sync_copy(tout, pairs_hbm.at[pl.ds(c0 // 2, TAIL_COLS // 2)])

    return reformat_kernel


def _make_gather():
    mesh = plsc.VectorSubcoreMesh(
        core_axis_name="c", subcore_axis_name="s", num_cores=NC, num_subcores=NS
    )

    @functools.partial(
        pl.kernel,
        out_type=jax.ShapeDtypeStruct((B, D), jnp.float32),
        mesh=mesh,
        compiler_params=pltpu.CompilerParams(
            needs_layout_passes=False, disable_bounds_checks=True),
        scratch_types=[
            pltpu.VMEM((IDX_PER_CHUNK,), jnp.int32),
            pltpu.VMEM((IDX_PER_CHUNK,), jnp.int32),
            pltpu.VMEM((IDX_PER_CHUNK, 2 * D), jnp.float32),
            pltpu.VMEM((CHUNK, D), jnp.float32),
            pltpu.SemaphoreType.DMA,
        ],
    )
    def cbow_kernel(g_hbm, o_hbm, table_hbm, out_hbm, idx_v, off_v,
                    rows_v, out_v, sem):
        wid = lax.axis_index("s") * NC + lax.axis_index("c")
        base = wid * ROWS_PER_W

        def chunk_body(ci, carry):
            cbase = base + ci * CHUNK
            pltpu.sync_copy(g_hbm.at[pl.ds(cbase * CTX, IDX_PER_CHUNK)], idx_v)
            pltpu.sync_copy(o_hbm.at[pl.ds(cbase * CTX, IDX_PER_CHUNK)], off_v)
            descs = [
                pltpu.async_copy(
                    table_hbm.at[idx_v.at[pl.ds(g * G, G)]],
                    rows_v.at[pl.ds(g * G, G)],
                    sem,
                )
                for g in range(NG)
            ]
            for d in descs:
                d.wait()

            lanes = lax.iota(jnp.int32, L)

            @plsc.parallel_loop(0, CHUNK, step=1, unroll=2)
            def _red(b):
                r0 = b * CTX
                accs = [jnp.zeros((L,), jnp.float32) for _ in range(D // L)]
                for j in range(CTX):
                    row = jnp.full((L,), r0 + j, jnp.int32)
                    off = plsc.load_gather(off_v, [row])
                    col0 = off + lanes
                    for k in range(D // L):
                        accs[k] = accs[k] + plsc.load_gather(
                            rows_v, [row, col0 + (k * L)])
                for k in range(D // L):
                    out_v[b, pl.ds(k * L, L)] = accs[k] * jnp.float32(1.0 / CTX)
            pltpu.sync_copy(out_v, out_hbm.at[pl.ds(cbase, CHUNK)])
            return carry

        lax.fori_loop(0, N_CHUNKS, chunk_body, 0)

    return cbow_kernel


_reformat = _make_reformat()
_cbow = _make_gather()


@jax.jit
def kernel(x, y, table):
    del y
    x_flat = x.astype(jnp.int32).reshape(B * CTX)
    g_flat = x_flat >> 1
    o_flat = (x_flat & 1) * D
    table_pairs = _reformat(table.T)
    return _cbow(g_flat, o_flat, table_pairs)


# speedup vs baseline: 1.3278x; 1.3278x over previous
import functools

import jax
import jax.numpy as jnp
from jax import lax
from jax.experimental import pallas as pl
from jax.experimental.pallas import tpu as pltpu
from jax.experimental.pallas import tpu_sc as plsc

B = 16384
CTX = 20
D = 64
L = 16
NC = 2
NS = 16
NW = NC * NS
ROWS_PER_W = B // NW
CHUNK = 32
N_CHUNKS = ROWS_PER_W // CHUNK
IDX_PER_CHUNK = CHUNK * CTX
G = 128
NG = IDX_PER_CHUNK // G


def _make_kernel():
    mesh = plsc.VectorSubcoreMesh(
        core_axis_name="c", subcore_axis_name="s", num_cores=NC, num_subcores=NS
    )

    @functools.partial(
        pl.kernel,
        out_type=jax.ShapeDtypeStruct((B, D), jnp.float32),
        mesh=mesh,
        compiler_params=pltpu.CompilerParams(use_tc_tiling_on_sc=False),
        scratch_types=[
            pltpu.VMEM((IDX_PER_CHUNK,), jnp.int32),
            pltpu.VMEM((IDX_PER_CHUNK, D), jnp.float32),
            pltpu.VMEM((CHUNK, D), jnp.float32),
            pltpu.SemaphoreType.DMA,
        ],
    )
    def cbow_kernel(x_hbm, table_hbm, out_hbm, idx_v, rows_v, out_v, sem):
        wid = lax.axis_index("s") * NC + lax.axis_index("c")
        base = wid * ROWS_PER_W

        def chunk_body(ci, carry):
            cbase = base + ci * CHUNK
            pltpu.sync_copy(x_hbm.at[pl.ds(cbase * CTX, IDX_PER_CHUNK)], idx_v)
            descs = [
                pltpu.async_copy(
                    table_hbm.at[idx_v.at[pl.ds(g * G, G)]],
                    rows_v.at[pl.ds(g * G, G)],
                    sem,
                )
                for g in range(NG)
            ]
            for d in descs:
                d.wait()

            def red_body(b, carry2):
                r0 = b * CTX
                for k in range(D // L):
                    acc = rows_v[r0, pl.ds(k * L, L)]
                    for j in range(1, CTX):
                        acc = acc + rows_v[r0 + j, pl.ds(k * L, L)]
                    out_v[b, pl.ds(k * L, L)] = acc * jnp.float32(1.0 / CTX)
                return carry2

            lax.fori_loop(0, CHUNK, red_body, 0)
            pltpu.sync_copy(out_v, out_hbm.at[pl.ds(cbase, CHUNK)])
            return carry

        lax.fori_loop(0, N_CHUNKS, chunk_body, 0)

    return cbow_kernel


_cbow = _make_kernel()


@jax.jit
def kernel(x, y, table):
    del y
    x_flat = x.astype(jnp.int32).reshape(B * CTX)
    return _cbow(x_flat, table)


# speedup vs baseline: 1.4469x; 1.0897x over previous
import functools

import jax
import jax.numpy as jnp
from jax import lax
from jax.experimental import pallas as pl
from jax.experimental.pallas import tpu as pltpu
from jax.experimental.pallas import tpu_sc as plsc

B = 16384
CTX = 20
D = 64
L = 16
NC = 2
NS = 16
NW = NC * NS
ROWS_PER_W = B // NW
CHUNK = 32
N_CHUNKS = ROWS_PER_W // CHUNK
IDX_PER_CHUNK = CHUNK * CTX
G = 128
NG = IDX_PER_CHUNK // G


def _make_kernel():
    mesh = plsc.VectorSubcoreMesh(
        core_axis_name="c", subcore_axis_name="s", num_cores=NC, num_subcores=NS
    )

    @functools.partial(
        pl.kernel,
        out_type=jax.ShapeDtypeStruct((B, D), jnp.float32),
        mesh=mesh,
        compiler_params=pltpu.CompilerParams(use_tc_tiling_on_sc=False),
        scratch_types=[
            pltpu.VMEM((IDX_PER_CHUNK,), jnp.int32),
            pltpu.VMEM((IDX_PER_CHUNK,), jnp.int32),
            pltpu.VMEM((IDX_PER_CHUNK, D), jnp.float32),
            pltpu.VMEM((IDX_PER_CHUNK, D), jnp.float32),
            pltpu.VMEM((CHUNK, D), jnp.float32),
            pltpu.SemaphoreType.DMA,
            pltpu.SemaphoreType.DMA,
        ],
    )
    def cbow_kernel(x_hbm, table_hbm, out_hbm, idx0, idx1, rows0, rows1,
                    out_v, sem0, sem1):
        wid = lax.axis_index("s") * NC + lax.axis_index("c")
        base = wid * ROWS_PER_W
        idxs, rows, sems = (idx0, idx1), (rows0, rows1), (sem0, sem1)

        def stage_and_fire(ci, par):
            cbase = base + ci * CHUNK
            pltpu.sync_copy(x_hbm.at[pl.ds(cbase * CTX, IDX_PER_CHUNK)],
                            idxs[par])
            for g in range(NG):
                pltpu.async_copy(
                    table_hbm.at[idxs[par].at[pl.ds(g * G, G)]],
                    rows[par].at[pl.ds(g * G, G)],
                    sems[par],
                )

        def drain(par):
            for g in range(NG):
                pltpu.make_async_copy(
                    table_hbm.at[idxs[par].at[pl.ds(g * G, G)]],
                    rows[par].at[pl.ds(g * G, G)],
                    sems[par],
                ).wait()

        stage_and_fire(0, 0)
        stage_and_fire(1, 1)

        def chunk_pair(gi, carry):
            for par in range(2):
                ci = 2 * gi + par
                cbase = base + ci * CHUNK
                drain(par)
                rows_v = rows[par]

                @plsc.parallel_loop(0, CHUNK, step=1, unroll=2)
                def _red(b):
                    r0 = b * CTX
                    for k in range(D // L):
                        acc = rows_v[r0, pl.ds(k * L, L)]
                        for j in range(1, CTX):
                            acc = acc + rows_v[r0 + j, pl.ds(k * L, L)]
                        out_v[b, pl.ds(k * L, L)] = acc * jnp.float32(1.0 / CTX)

                pltpu.sync_copy(out_v, out_hbm.at[pl.ds(cbase, CHUNK)])

                @pl.when(ci + 2 < N_CHUNKS)
                def _prefetch():
                    stage_and_fire(ci + 2, par)
            return carry

        lax.fori_loop(0, N_CHUNKS // 2, chunk_pair, 0)

    return cbow_kernel


_cbow = _make_kernel()


@jax.jit
def kernel(x, y, table):
    del y
    x_flat = x.astype(jnp.int32).reshape(B * CTX)
    return _cbow(x_flat, table)
